# initial kernel scaffold (unmeasured)
import jax
import jax.numpy as jnp
from jax import lax
from jax.experimental import pallas as pl
from jax.experimental.pallas import tpu as pltpu


def kernel(
    x,
):
    def body(*refs):
        pass

    out_shape = jax.ShapeDtypeStruct(..., jnp.float32)
    return pl.pallas_call(body, out_shape=out_shape)(...)


# baseline (device time: 997064 ns/iter reference)
import functools

import jax
import jax.numpy as jnp
from jax import lax
from jax.experimental import pallas as pl
from jax.experimental.pallas import tpu as pltpu

M = 32768
N = 1024
CHUNK = 2048
NC = M // CHUNK


def kernel(x):
    x_bf = x.astype(jnp.bfloat16)

    def body(x_ref, out_ref, recv_ref, va, vb, vo, local_sems, send_sem,
             recv_sem):
        mx = lax.axis_index("x")
        my = lax.axis_index("y")
        mz = lax.axis_index("z")
        partner = (1 - mx, my, mz)

        barrier = pltpu.get_barrier_semaphore()
        pl.semaphore_signal(barrier, 1, device_id=partner,
                            device_id_type=pl.DeviceIdType.MESH)
        pl.semaphore_wait(barrier, 1)

        rdma = pltpu.make_async_remote_copy(
            src_ref=x_ref,
            dst_ref=recv_ref,
            send_sem=send_sem,
            recv_sem=recv_sem,
            device_id=partner,
            device_id_type=pl.DeviceIdType.MESH,
        )
        rdma.start()
        rdma.wait()

        for c in range(NC):
            sl = pl.ds(c * CHUNK, CHUNK)
            cp_a = pltpu.make_async_copy(x_ref.at[sl], va, local_sems.at[0])
            cp_b = pltpu.make_async_copy(recv_ref.at[sl], vb, local_sems.at[1])
            cp_a.start()
            cp_b.start()
            cp_a.wait()
            cp_b.wait()
            vo[...] = va[...].astype(jnp.float32) + vb[...].astype(jnp.float32)
            cp_o = pltpu.make_async_copy(vo, out_ref.at[sl], local_sems.at[2])
            cp_o.start()
            cp_o.wait()

        @functools.partial(pl.run_scoped, sem2=pltpu.SemaphoreType.REGULAR)
        def _(sem2):
            pl.semaphore_signal(sem2, 1, device_id=partner,
                                device_id_type=pl.DeviceIdType.MESH)
            pl.semaphore_wait(sem2, 1)

    out, _recv = pl.pallas_call(
        body,
        out_shape=[
            jax.ShapeDtypeStruct((M, N), jnp.float32),
            jax.ShapeDtypeStruct((M, N), jnp.bfloat16),
        ],
        in_specs=[pl.BlockSpec(memory_space=pl.ANY)],
        out_specs=[
            pl.BlockSpec(memory_space=pl.ANY),
            pl.BlockSpec(memory_space=pl.ANY),
        ],
        scratch_shapes=[
            pltpu.VMEM((CHUNK, N), jnp.bfloat16),
            pltpu.VMEM((CHUNK, N), jnp.bfloat16),
            pltpu.VMEM((CHUNK, N), jnp.float32),
            pltpu.SemaphoreType.DMA((3,)),
            pltpu.SemaphoreType.DMA,
            pltpu.SemaphoreType.DMA,
        ],
        compiler_params=pltpu.CompilerParams(collective_id=0),
    )(x_bf)
    return out
